# Initial kernel scaffold; baseline (speedup 1.0000x reference)
import jax
import jax.numpy as jnp
from jax.experimental import pallas as pl


def kernel(x, rand_proj, codebook, values):
    raise NotImplementedError("write your pallas kernel here")


# speedup vs baseline: 4.7783x; 4.7783x over previous
import functools

import jax
import jax.numpy as jnp
from jax import lax
from jax.experimental import pallas as pl
from jax.experimental.pallas import tpu as pltpu
from jax.experimental.pallas import tpu_sc as plsc

B, N, D = 8, 1024, 256
H, K, DM = 4, 1024, 256


def _dist_argmin_body(x_ref, rp_ref, cb_ref, idx_ref):
    hi = pl.program_id(1)
    x = x_ref[0]
    rp = rp_ref[0]
    cb = cb_ref[0]
    xp = jax.lax.dot_general(x, rp, (((1,), (0,)), ((), ())),
                             preferred_element_type=jnp.float32)
    dots = jax.lax.dot_general(xp, cb, (((1,), (1,)), ((), ())),
                               preferred_element_type=jnp.float32)
    x2 = jnp.sum(xp * xp, axis=1, keepdims=True)
    e2 = jnp.sum(cb * cb, axis=1)
    dist = x2 - 2.0 * dots + e2[None, :]
    m = jnp.min(dist, axis=1, keepdims=True)
    kiota = lax.broadcasted_iota(jnp.int32, (N, K), 1)
    cand = jnp.where(dist <= m, kiota, K)
    idx = jnp.min(cand, axis=1).astype(jnp.int32)
    idx_ref[0, 0, 0] = idx + hi * K


def _dist_argmin(x, rand_proj, codebook):
    return pl.pallas_call(
        _dist_argmin_body,
        grid=(B, H),
        in_specs=[
            pl.BlockSpec((1, N, D), lambda bi, hi: (bi, 0, 0)),
            pl.BlockSpec((1, D, D), lambda bi, hi: (hi, 0, 0)),
            pl.BlockSpec((1, K, D), lambda bi, hi: (hi, 0, 0)),
        ],
        out_specs=pl.BlockSpec((1, 1, 1, N), lambda bi, hi: (bi, hi, 0, 0)),
        out_shape=jax.ShapeDtypeStruct((B, H, 1, N), jnp.int32),
    )(x, rand_proj, codebook)


try:
    _info = plsc.get_sparse_core_info()
    _NC, _NS, _L = _info.num_cores, _info.num_subcores, _info.num_lanes
except ValueError:
    _NC, _NS, _L = 2, 16, 16
_NW = _NC * _NS
_POS_PER_W = (B * N) // _NW
_CS = 32
_NCHUNK = _POS_PER_W // _CS


def _sc_gather_mean(idx_g, vals_flat):
    mesh = plsc.VectorSubcoreMesh(core_axis_name="c", subcore_axis_name="s")

    @functools.partial(
        pl.kernel, mesh=mesh,
        out_type=jax.ShapeDtypeStruct((B, N, DM), jnp.float32),
        scratch_types=[
            pltpu.VMEM((H * _CS,), jnp.int32),
            pltpu.VMEM((H * _CS, DM), jnp.float32),
            pltpu.VMEM((_CS, DM), jnp.float32),
            pltpu.SemaphoreType.DMA,
        ],
    )
    def k(idx_hbm, table_hbm, out_hbm, idx_v, rows_v, acc_v, sem):
        wid = lax.axis_index("s") * _NC + lax.axis_index("c")
        b = wid // (N // _POS_PER_W)
        n0 = (wid % (N // _POS_PER_W)) * _POS_PER_W
        for j in range(_NCHUNK):
            nj = n0 + j * _CS
            for h in range(H):
                pltpu.sync_copy(idx_hbm.at[b, h, pl.ds(nj, _CS)],
                                idx_v.at[pl.ds(h * _CS, _CS)])
            pltpu.async_copy(table_hbm.at[idx_v], rows_v, sem).wait()

            def acc_row(r, _):
                for v in range(DM // _L):
                    sl = pl.ds(v * _L, _L)
                    s = (rows_v[r, sl] + rows_v[_CS + r, sl]
                         + rows_v[2 * _CS + r, sl] + rows_v[3 * _CS + r, sl])
                    acc_v[r, sl] = s * 0.25
                return 0

            lax.fori_loop(0, _CS, acc_row, 0)
            pltpu.sync_copy(acc_v, out_hbm.at[b, pl.ds(nj, _CS)])

    return k(idx_g, vals_flat)


def kernel(x, rand_proj, codebook, values):
    idx_g = _dist_argmin(x, rand_proj, codebook)
    idx_g = idx_g.reshape(B, H, N)
    vals_flat = values.reshape(H * K, DM)
    return _sc_gather_mean(idx_g, vals_flat)


# speedup vs baseline: 4.8577x; 1.0166x over previous
import functools

import jax
import jax.numpy as jnp
from jax import lax
from jax.experimental import pallas as pl
from jax.experimental.pallas import tpu as pltpu
from jax.experimental.pallas import tpu_sc as plsc

B, N, D = 8, 1024, 256
H, K, DM = 4, 1024, 256


def _dist_argmin_body(x_ref, rp_ref, cb_ref, idx_ref):
    hi = pl.program_id(1)
    x = x_ref[0]
    rp = rp_ref[0]
    cb = cb_ref[0]
    xp = jax.lax.dot_general(x, rp, (((1,), (0,)), ((), ())),
                             preferred_element_type=jnp.float32)
    dots = jax.lax.dot_general(xp, cb, (((1,), (1,)), ((), ())),
                               preferred_element_type=jnp.float32)
    x2 = jnp.sum(xp * xp, axis=1, keepdims=True)
    e2 = jnp.sum(cb * cb, axis=1)
    dist = x2 - 2.0 * dots + e2[None, :]
    m = jnp.min(dist, axis=1, keepdims=True)
    kiota = lax.broadcasted_iota(jnp.int32, (N, K), 1)
    cand = jnp.where(dist <= m, kiota, K)
    idx = jnp.min(cand, axis=1).astype(jnp.int32)
    idx_ref[0, 0, 0] = idx + hi * K


def _dist_argmin(x, rand_proj, codebook):
    return pl.pallas_call(
        _dist_argmin_body,
        grid=(B, H),
        in_specs=[
            pl.BlockSpec((1, N, D), lambda bi, hi: (bi, 0, 0)),
            pl.BlockSpec((1, D, D), lambda bi, hi: (hi, 0, 0)),
            pl.BlockSpec((1, K, D), lambda bi, hi: (hi, 0, 0)),
        ],
        out_specs=pl.BlockSpec((1, 1, 1, N), lambda bi, hi: (bi, hi, 0, 0)),
        out_shape=jax.ShapeDtypeStruct((B, H, 1, N), jnp.int32),
    )(x, rand_proj, codebook)


try:
    _info = plsc.get_sparse_core_info()
    _NC, _NS, _L = _info.num_cores, _info.num_subcores, _info.num_lanes
except ValueError:
    _NC, _NS, _L = 2, 16, 16
_NW = _NC * _NS
_POS_PER_W = (B * N) // _NW
_CS = 32
_NCHUNK = _POS_PER_W // _CS


def _sc_gather_mean(idx_g, vals_flat):
    mesh = plsc.VectorSubcoreMesh(core_axis_name="c", subcore_axis_name="s")

    @functools.partial(
        pl.kernel, mesh=mesh,
        out_type=jax.ShapeDtypeStruct((B, N, DM), jnp.float32),
        scratch_types=[
            pltpu.VMEM((H, _POS_PER_W), jnp.int32),
            pltpu.VMEM((2, H * _CS, DM), jnp.float32),
            pltpu.VMEM((2, _CS, DM), jnp.float32),
            pltpu.SemaphoreType.DMA,
            pltpu.SemaphoreType.DMA,
            pltpu.SemaphoreType.DMA,
            pltpu.SemaphoreType.DMA,
        ],
    )
    def k(idx_hbm, table_hbm, out_hbm, idx_v, rows_v, acc_v,
          gsem0, gsem1, osem0, osem1):
        gsems = (gsem0, gsem1)
        osems = (osem0, osem1)
        wid = lax.axis_index("s") * _NC + lax.axis_index("c")
        b = wid // (N // _POS_PER_W)
        n0 = (wid % (N // _POS_PER_W)) * _POS_PER_W

        pltpu.async_copy(idx_hbm.at[b, :, pl.ds(n0, _POS_PER_W)],
                         idx_v, gsem0).wait()

        def gather_args(j, buf):
            for h in range(H):
                yield (table_hbm.at[idx_v.at[h, pl.ds(j * _CS, _CS)]],
                       rows_v.at[buf, pl.ds(h * _CS, _CS)], gsems[buf])

        for a in gather_args(0, 0):
            pltpu.async_copy(*a)

        for j in range(_NCHUNK):
            buf = j % 2
            if j + 1 < _NCHUNK:
                for a in gather_args(j + 1, 1 - buf):
                    pltpu.async_copy(*a)
            for a in gather_args(j, buf):
                pltpu.make_async_copy(*a).wait()
            if j >= 2:
                pltpu.make_async_copy(
                    acc_v.at[buf],
                    out_hbm.at[b, pl.ds(n0 + (j - 2) * _CS, _CS)],
                    osems[buf]).wait()

            def acc_row(r, _):
                for v in range(DM // _L):
                    sl = pl.ds(v * _L, _L)
                    s = (rows_v[buf, r, sl] + rows_v[buf, _CS + r, sl]
                         + rows_v[buf, 2 * _CS + r, sl]
                         + rows_v[buf, 3 * _CS + r, sl])
                    acc_v[buf, r, sl] = s * 0.25
                return 0

            lax.fori_loop(0, _CS, acc_row, 0)
            pltpu.async_copy(acc_v.at[buf],
                             out_hbm.at[b, pl.ds(n0 + j * _CS, _CS)],
                             osems[buf])

        for j in (_NCHUNK - 2, _NCHUNK - 1):
            buf = j % 2
            pltpu.make_async_copy(
                acc_v.at[buf],
                out_hbm.at[b, pl.ds(n0 + j * _CS, _CS)],
                osems[buf]).wait()

    return k(idx_g, vals_flat)


def kernel(x, rand_proj, codebook, values):
    idx_g = _dist_argmin(x, rand_proj, codebook)
    idx_g = idx_g.reshape(B, H, N)
    vals_flat = values.reshape(H * K, DM)
    return _sc_gather_mean(idx_g, vals_flat)
